# baseline (speedup 1.0000x reference)
import functools

import jax
import jax.numpy as jnp
from jax import lax
from jax.experimental import pallas as pl
from jax.experimental.pallas import tpu as pltpu
from jax.experimental.pallas import tpu_sc as plsc

B = 16384
DIM = 64
NENT = 1000001
FLAT = 3 * B
NW = 32
ROWS_W = FLAT // NW
CH = 128
NCH = ROWS_W // CH
BE = 4096
GT = 245
NPAD = GT * BE


def _tc_transpose_table(ent_t):
    def body(x_ref, o_ref):
        o_ref[...] = x_ref[...].T

    return pl.pallas_call(
        body,
        grid=(GT,),
        in_specs=[pl.BlockSpec((DIM, BE), lambda i: (0, i))],
        out_specs=pl.BlockSpec((BE, DIM), lambda i: (i, 0)),
        out_shape=jax.ShapeDtypeStruct((NPAD, DIM), jnp.float32),
    )(ent_t)


def _sc_gather_combine(ent_conv, rel_emb, sub_i, obj_i, rel_i):
    mesh = plsc.VectorSubcoreMesh(core_axis_name="c", subcore_axis_name="s")

    @functools.partial(
        pl.kernel,
        mesh=mesh,
        out_type=jax.ShapeDtypeStruct((FLAT, DIM), jnp.float32),
        scratch_types=[
            pltpu.VMEM((NCH, CH), jnp.int32),
            pltpu.VMEM((NCH, CH), jnp.int32),
            pltpu.VMEM((NCH, CH), jnp.int32),
            pltpu.VMEM((CH, DIM), jnp.float32),
            pltpu.VMEM((CH, DIM), jnp.float32),
            pltpu.VMEM((CH, DIM), jnp.float32),
            pltpu.SemaphoreType.DMA,
            pltpu.SemaphoreType.DMA,
            pltpu.SemaphoreType.DMA,
        ],
        compiler_params=pltpu.CompilerParams(use_tc_tiling_on_sc=False),
    )
    def k(ent_hbm, rel_hbm, sub_hbm, obj_hbm, reli_hbm, out_hbm,
          idx_s, idx_o, idx_r, buf_s, buf_o, buf_r, sem_s, sem_o, sem_r):
        wid = lax.axis_index("s") * 2 + lax.axis_index("c")
        blk0 = wid * NCH
        pltpu.sync_copy(sub_hbm.at[wid], idx_s)
        pltpu.sync_copy(obj_hbm.at[wid], idx_o)
        pltpu.sync_copy(reli_hbm.at[wid], idx_r)

        def chunk(j, carry):
            cs = pltpu.async_copy(ent_hbm.at[idx_s.at[j]], buf_s, sem_s)
            co = pltpu.async_copy(ent_hbm.at[idx_o.at[j]], buf_o, sem_o)
            cr = pltpu.async_copy(rel_hbm.at[idx_r.at[j]], buf_r, sem_r)
            cs.wait()
            co.wait()
            cr.wait()

            def row(rr, c2):
                for c4 in range(DIM // 16):
                    sl = pl.ds(c4 * 16, 16)
                    buf_s[rr, sl] = buf_s[rr, sl] + buf_r[rr, sl] - buf_o[rr, sl]
                return c2
            lax.fori_loop(0, CH, row, 0)
            pltpu.sync_copy(buf_s, out_hbm.at[pl.ds((blk0 + j) * CH, CH)])
            return carry
        lax.fori_loop(0, NCH, chunk, 0)

    return k(ent_conv, rel_emb, sub_i, obj_i, rel_i)


def _tc_score(d, W, b2):
    BLK = 2048

    def body(d_ref, w_ref, b_ref, o_ref):
        y = lax.dot_general(d_ref[...], w_ref[...],
                            (((1,), (1,)), ((), ())),
                            preferred_element_type=jnp.float32)
        y = y + b_ref[...]
        o_ref[...] = jnp.sum(y * y, axis=1, keepdims=True)

    return pl.pallas_call(
        body,
        grid=(B // BLK,),
        in_specs=[
            pl.BlockSpec((BLK, 3 * DIM), lambda i: (i, 0)),
            pl.BlockSpec((DIM, 3 * DIM), lambda i: (0, 0)),
            pl.BlockSpec((1, DIM), lambda i: (0, 0)),
        ],
        out_specs=pl.BlockSpec((BLK, 1), lambda i: (i, 0)),
        out_shape=jax.ShapeDtypeStruct((B, 1), jnp.float32),
    )(d, W, b2)


def kernel(subjects, objects, relations, ent_emb, rel_emb, W, b):
    ent_conv = _tc_transpose_table(jnp.transpose(ent_emb))
    sub_i = subjects.reshape(NW, NCH, CH)
    obj_i = objects.reshape(NW, NCH, CH)
    rel_i = relations.reshape(NW, NCH, CH)
    d = _sc_gather_combine(ent_conv, rel_emb, sub_i, obj_i, rel_i)
    return _tc_score(d.reshape(B, 3 * DIM), W, b.reshape(1, DIM))


# speedup vs baseline: 1.5382x; 1.5382x over previous
import functools

import jax
import jax.numpy as jnp
from jax import lax
from jax.experimental import pallas as pl
from jax.experimental.pallas import tpu as pltpu
from jax.experimental.pallas import tpu_sc as plsc

B = 16384
DIM = 64
NENT = 1000001
FLAT = 3 * B
NW = 32
ROWS_W = FLAT // NW
CH = 128
NCH = ROWS_W // CH
BE = 4096
GT = 245
NPAD = GT * BE


def _tc_transpose_table(ent_t):
    def body(x_ref, o_ref):
        t = x_ref[...].T
        o_ref[...] = jnp.concatenate([t, t], axis=1)

    return pl.pallas_call(
        body,
        grid=(GT,),
        in_specs=[pl.BlockSpec((DIM, BE), lambda i: (0, i))],
        out_specs=pl.BlockSpec((BE, 2 * DIM), lambda i: (i, 0)),
        out_shape=jax.ShapeDtypeStruct((NPAD, 2 * DIM), jnp.float32),
    )(ent_t)


def _sc_gather_combine(ent_conv, rel_emb, sub_i, obj_i, rel_i):
    mesh = plsc.VectorSubcoreMesh(core_axis_name="c", subcore_axis_name="s")

    @functools.partial(
        pl.kernel,
        mesh=mesh,
        out_type=jax.ShapeDtypeStruct((FLAT, DIM), jnp.float32),
        scratch_types=[
            pltpu.VMEM((NCH, CH), jnp.int32),
            pltpu.VMEM((NCH, CH), jnp.int32),
            pltpu.VMEM((NCH, CH), jnp.int32),
            pltpu.VMEM((CH, 2 * DIM), jnp.float32),
            pltpu.VMEM((CH, 2 * DIM), jnp.float32),
            pltpu.VMEM((CH, DIM), jnp.float32),
            pltpu.SemaphoreType.DMA,
            pltpu.SemaphoreType.DMA,
            pltpu.SemaphoreType.DMA,
        ],
        compiler_params=pltpu.CompilerParams(use_tc_tiling_on_sc=False),
    )
    def k(ent_hbm, rel_hbm, sub_hbm, obj_hbm, reli_hbm, out_hbm,
          idx_s, idx_o, idx_r, buf_s, buf_o, buf_r, sem_s, sem_o, sem_r):
        wid = lax.axis_index("s") * 2 + lax.axis_index("c")
        blk0 = wid * NCH
        pltpu.sync_copy(sub_hbm.at[wid], idx_s)
        pltpu.sync_copy(obj_hbm.at[wid], idx_o)
        pltpu.sync_copy(reli_hbm.at[wid], idx_r)

        def chunk(j, carry):
            cs = pltpu.async_copy(ent_hbm.at[idx_s.at[j]], buf_s, sem_s)
            co = pltpu.async_copy(ent_hbm.at[idx_o.at[j]], buf_o, sem_o)
            cr = pltpu.async_copy(rel_hbm.at[idx_r.at[j]], buf_r, sem_r)
            cs.wait()
            co.wait()
            cr.wait()

            def row(rr, c2):
                for c4 in range(DIM // 16):
                    sl = pl.ds(c4 * 16, 16)
                    buf_r[rr, sl] = buf_s[rr, sl] + buf_r[rr, sl] - buf_o[rr, sl]
                return c2
            lax.fori_loop(0, CH, row, 0)
            pltpu.sync_copy(buf_r, out_hbm.at[pl.ds((blk0 + j) * CH, CH)])
            return carry
        lax.fori_loop(0, NCH, chunk, 0)

    return k(ent_conv, rel_emb, sub_i, obj_i, rel_i)


def _tc_score(d, W, b2):
    BLK = 2048

    def body(d_ref, w_ref, b_ref, o_ref):
        y = lax.dot_general(d_ref[...], w_ref[...],
                            (((1,), (1,)), ((), ())),
                            preferred_element_type=jnp.float32)
        y = y + b_ref[...]
        o_ref[...] = jnp.sum(y * y, axis=1, keepdims=True)

    return pl.pallas_call(
        body,
        grid=(B // BLK,),
        in_specs=[
            pl.BlockSpec((BLK, 3 * DIM), lambda i: (i, 0)),
            pl.BlockSpec((DIM, 3 * DIM), lambda i: (0, 0)),
            pl.BlockSpec((1, DIM), lambda i: (0, 0)),
        ],
        out_specs=pl.BlockSpec((BLK, 1), lambda i: (i, 0)),
        out_shape=jax.ShapeDtypeStruct((B, 1), jnp.float32),
    )(d, W, b2)


def kernel(subjects, objects, relations, ent_emb, rel_emb, W, b):
    ent_conv = _tc_transpose_table(jnp.transpose(ent_emb))
    sub_i = subjects.reshape(NW, NCH, CH)
    obj_i = objects.reshape(NW, NCH, CH)
    rel_i = relations.reshape(NW, NCH, CH)
    d = _sc_gather_combine(ent_conv, rel_emb, sub_i, obj_i, rel_i)
    return _tc_score(d.reshape(B, 3 * DIM), W, b.reshape(1, DIM))
